# baseline (speedup 1.0000x reference)
import functools

import jax
import jax.numpy as jnp
import numpy as np
from jax import lax
from jax.experimental import pallas as pl
from jax.experimental.pallas import tpu as pltpu
from jax.experimental.pallas import tpu_sc as plsc

N = 10000
E = 160000
D = 256
NSEED = 1024
H = 128

NC = 2
NS = 16
NW = NC * NS

EPAD = 163840
CE = 128
CEB = 80
NS1 = 4
CEC = 16
NSLOT = 4
NROW = 10112
ZROW = NROW // NS
BROW = 1040
ZROWB = BROW // NS

_mesh = plsc.VectorSubcoreMesh(
    core_axis_name="c", subcore_axis_name="s", num_cores=NC, num_subcores=NS)


@functools.partial(
    pl.kernel,
    out_type=(
        jax.ShapeDtypeStruct((NC, NROW, H), jnp.float32),
    ),
    mesh=_mesh,
    scratch_types=(
        [pltpu.VMEM_SHARED((NROW, H), jnp.float32)]
        + [pltpu.VMEM((CEB, H), jnp.float32)] * NS1
        + [pltpu.VMEM((CEB,), jnp.int32)] * (2 * NS1)
        + [pltpu.SemaphoreType.DMA] * (3 * NS1)
    ),
    compiler_params=pltpu.CompilerParams(needs_layout_passes=False),
)
def _agg1_kernel(src_hbm, dst_hbm, t0_hbm, t1_hbm, zrow_hbm,
                 agg_hbm, acc_sh, *rest):
    rows_v = rest[:NS1]
    src_v = rest[NS1:2 * NS1]
    dst_v = rest[2 * NS1:3 * NS1]
    sem_i = rest[3 * NS1:4 * NS1]
    sem_g = rest[4 * NS1:5 * NS1]
    sem_s = rest[5 * NS1:6 * NS1]
    c = lax.axis_index("c")
    s = lax.axis_index("s")
    pltpu.sync_copy(zrow_hbm, acc_sh.at[pl.ds(s * ZROW, ZROW)])
    plsc.subcore_barrier()

    nchunk = (EPAD // NS) // CEB
    ebase = s * (EPAD // NS)

    def srcat(k):
        return src_hbm.at[pl.ds(ebase + k * CEB, CEB)]

    def dstat(k):
        return dst_hbm.at[pl.ds(ebase + k * CEB, CEB)]

    def run(table_hbm):
        for b in range(NS1):
            pltpu.sync_copy(srcat(b), src_v[b])
            pltpu.sync_copy(dstat(b), dst_v[b])
            pltpu.async_copy(table_hbm.at[src_v[b]], rows_v[b], sem_g[b])

        def body(jj, carry):
            for b in range(NS1):
                pltpu.make_async_copy(table_hbm.at[src_v[b]], rows_v[b],
                                      sem_g[b]).wait()
                pltpu.async_copy(rows_v[b], acc_sh.at[dst_v[b]], sem_s[b],
                                 add=True)

            @pl.when(jj < nchunk // NS1 - 1)
            def _():
                for b in range(NS1):
                    k = NS1 * jj + b + NS1
                    pltpu.make_async_copy(rows_v[b], acc_sh.at[dst_v[b]],
                                          sem_s[b]).wait()
                    pltpu.async_copy(srcat(k), src_v[b], sem_i[b])
                    pltpu.async_copy(dstat(k), dst_v[b], sem_i[b])
                for b in range(NS1):
                    k = NS1 * jj + b + NS1
                    pltpu.make_async_copy(srcat(k), src_v[b], sem_i[b]).wait()
                    pltpu.make_async_copy(dstat(k), dst_v[b], sem_i[b]).wait()
                    pltpu.async_copy(table_hbm.at[src_v[b]], rows_v[b],
                                     sem_g[b])
            return carry

        lax.fori_loop(0, nchunk // NS1, body, 0)
        for b in range(NS1):
            pltpu.make_async_copy(rows_v[b], acc_sh.at[dst_v[b]],
                                  sem_s[b]).wait()

    @pl.when(c == 0)
    def _():
        run(t0_hbm)

    @pl.when(c == 1)
    def _():
        run(t1_hbm)

    plsc.subcore_barrier()
    pltpu.sync_copy(acc_sh.at[pl.ds(s * ZROW, ZROW)],
                    agg_hbm.at[c, pl.ds(s * ZROW, ZROW)])


PERW = EPAD // NW


@functools.partial(
    pl.kernel,
    out_type=(
        jax.ShapeDtypeStruct((NC, NROW, H), jnp.float32),
        jax.ShapeDtypeStruct((EPAD,), jnp.int32),
        jax.ShapeDtypeStruct((EPAD,), jnp.int32),
        jax.ShapeDtypeStruct((NW * 16,), jnp.int32),
    ),
    mesh=_mesh,
    scratch_types=(
        [pltpu.VMEM_SHARED((NROW, H), jnp.float32),
         pltpu.VMEM((CE, H), jnp.float32),
         pltpu.VMEM((PERW + 16,), jnp.int32),
         pltpu.VMEM((PERW + 16,), jnp.int32),
         pltpu.VMEM((16,), jnp.int32)]
        + [pltpu.VMEM((CE,), jnp.int32)] * (2 * NSLOT)
        + [pltpu.SemaphoreType.DMA] * (2 * NSLOT)
    ),
    compiler_params=pltpu.CompilerParams(needs_layout_passes=False),
)
def _deg_kernel(src_hbm, dst_hbm, zrow_hbm, ones_hbm, padz_hbm, padd_hbm,
                deg_hbm, csrc_hbm, cdst_hbm, cnt_hbm,
                acc_sh, ones_v, csrc_v, cdst_v, cnt_v, *rest):
    dst_v = rest[:NSLOT]
    src_v = rest[NSLOT:2 * NSLOT]
    sem_i = rest[2 * NSLOT:3 * NSLOT]
    sem_s = rest[3 * NSLOT:4 * NSLOT]
    c = lax.axis_index("c")
    s = lax.axis_index("s")
    w = s * NC + c
    pltpu.sync_copy(zrow_hbm, acc_sh.at[pl.ds(s * ZROW, ZROW)])
    pltpu.sync_copy(ones_hbm, ones_v)
    pltpu.sync_copy(padz_hbm, csrc_v.at[pl.ds(0, PERW + 16)])
    pltpu.sync_copy(padd_hbm, cdst_v.at[pl.ds(0, PERW + 16)])
    plsc.subcore_barrier()

    nchunk = PERW // CE
    ebase = w * PERW

    def didx_at(k):
        return dst_hbm.at[pl.ds(ebase + k * CE, CE)]

    def sidx_at(k):
        return src_hbm.at[pl.ds(ebase + k * CE, CE)]

    for b in range(NSLOT):
        pltpu.async_copy(didx_at(b), dst_v[b], sem_i[b])
        pltpu.async_copy(sidx_at(b), src_v[b], sem_i[b])

    def body(jj, cnt):
        for b in range(NSLOT):
            k = NSLOT * jj + b
            pltpu.make_async_copy(didx_at(k), dst_v[b], sem_i[b]).wait()
            pltpu.make_async_copy(sidx_at(k), src_v[b], sem_i[b]).wait()
            pltpu.async_copy(ones_v, acc_sh.at[dst_v[b]], sem_s[b], add=True)
            for g in range(CE // 16):
                dv = dst_v[b][pl.ds(g * 16, 16)]
                sv = src_v[b][pl.ds(g * 16, 16)]
                m = dv < NSEED
                dcl = jnp.where(m, dv, NSEED)
                dsrt, ssrt = plsc.sort_key_val(dcl, sv)
                cdst_v[pl.ds(cnt, 16)] = dsrt
                csrc_v[pl.ds(cnt, 16)] = ssrt
                cnt = cnt + jnp.sum(jnp.where(m, 1, 0))

        @pl.when(jj < nchunk // NSLOT - 1)
        def _():
            for b in range(NSLOT):
                k = NSLOT * jj + b
                pltpu.make_async_copy(ones_v, acc_sh.at[dst_v[b]],
                                      sem_s[b]).wait()
                pltpu.async_copy(didx_at(k + NSLOT), dst_v[b], sem_i[b])
                pltpu.async_copy(sidx_at(k + NSLOT), src_v[b], sem_i[b])
        return cnt

    cnt = lax.fori_loop(0, nchunk // NSLOT, body, jnp.int32(0))
    for b in range(NSLOT):
        pltpu.make_async_copy(ones_v, acc_sh.at[dst_v[b]], sem_s[b]).wait()
    pltpu.sync_copy(csrc_v.at[pl.ds(0, PERW)], csrc_hbm.at[pl.ds(ebase, PERW)])
    pltpu.sync_copy(cdst_v.at[pl.ds(0, PERW)], cdst_hbm.at[pl.ds(ebase, PERW)])
    nch = (cnt + (CEC - 1)) // CEC
    cnt_v[pl.ds(0, 16)] = jnp.broadcast_to(nch, (16,))
    pltpu.sync_copy(cnt_v, cnt_hbm.at[pl.ds(w * 16, 16)])
    plsc.subcore_barrier()
    pltpu.sync_copy(acc_sh.at[pl.ds(s * ZROW, ZROW)],
                    deg_hbm.at[c, pl.ds(s * ZROW, ZROW)])


@functools.partial(
    pl.kernel,
    out_type=(
        jax.ShapeDtypeStruct((NC, NSEED, H), jnp.float32),
        jax.ShapeDtypeStruct((NC, NSEED, H), jnp.float32),
    ),
    mesh=_mesh,
    scratch_types=(
        [pltpu.VMEM_SHARED((BROW, H), jnp.float32)] * 2
        + [pltpu.VMEM((CEC, H), jnp.float32)] * 4
        + [pltpu.VMEM((CEC,), jnp.int32)] * 4
        + [pltpu.VMEM((16,), jnp.int32)]
        + [pltpu.SemaphoreType.DMA] * 4
    ),
    compiler_params=pltpu.CompilerParams(needs_layout_passes=False),
)
def _agg2_kernel(src_hbm, dst_hbm, cnt_hbm, ta_hbm, tb_hbm, zrow_hbm,
                 pa_hbm, pb_hbm, acca_sh, accb_sh,
                 rowsa0_v, rowsb0_v, rowsa1_v, rowsb1_v,
                 src0_v, src1_v, dst0_v, dst1_v, cnt_v, sem0, sem1,
                 semx0, semx1):
    c = lax.axis_index("c")
    s = lax.axis_index("s")
    w = s * NC + c
    pltpu.sync_copy(zrow_hbm, acca_sh.at[pl.ds(s * ZROWB, ZROWB)])
    pltpu.sync_copy(zrow_hbm, accb_sh.at[pl.ds(s * ZROWB, ZROWB)])
    pltpu.sync_copy(cnt_hbm.at[pl.ds(w * 16, 16)], cnt_v)
    plsc.subcore_barrier()

    nch = jnp.maximum(jnp.max(cnt_v[...]), 2)
    ebase = w * PERW

    def load_idx(k, src_v, dst_v):
        base = ebase + k * CEC
        pltpu.sync_copy(src_hbm.at[pl.ds(base, CEC)], src_v)
        pltpu.sync_copy(dst_hbm.at[pl.ds(base, CEC)], dst_v)

    def fire(src_v, ra, rb, sem):
        pltpu.async_copy(ta_hbm.at[src_v], ra, sem)
        pltpu.async_copy(tb_hbm.at[src_v], rb, sem)

    def drain_scatter(ra, rb, dst_v, sem):
        pltpu.make_async_copy(ra, acca_sh.at[dst_v], sem).wait()
        pltpu.make_async_copy(rb, accb_sh.at[dst_v], sem).wait()

    load_idx(0, src0_v, dst0_v)
    fire(src0_v, rowsa0_v, rowsb0_v, sem0)
    load_idx(1, src1_v, dst1_v)
    fire(src1_v, rowsa1_v, rowsb1_v, sem1)

    def body(j, carry):
        k0 = 2 * j
        pltpu.make_async_copy(ta_hbm.at[src0_v], rowsa0_v, sem0).wait()
        pltpu.make_async_copy(tb_hbm.at[src0_v], rowsb0_v, sem0).wait()
        pltpu.async_copy(rowsa0_v, acca_sh.at[dst0_v], semx0, add=True)
        pltpu.async_copy(rowsb0_v, accb_sh.at[dst0_v], semx0, add=True)

        @pl.when(k0 + 2 < nch)
        def _():
            drain_scatter(rowsa0_v, rowsb0_v, dst0_v, semx0)
            load_idx(k0 + 2, src0_v, dst0_v)
            fire(src0_v, rowsa0_v, rowsb0_v, sem0)

        @pl.when(k0 + 1 < nch)
        def _():
            pltpu.make_async_copy(ta_hbm.at[src1_v], rowsa1_v, sem1).wait()
            pltpu.make_async_copy(tb_hbm.at[src1_v], rowsb1_v, sem1).wait()
            pltpu.async_copy(rowsa1_v, acca_sh.at[dst1_v], semx1, add=True)
            pltpu.async_copy(rowsb1_v, accb_sh.at[dst1_v], semx1, add=True)

            @pl.when(k0 + 3 < nch)
            def _():
                drain_scatter(rowsa1_v, rowsb1_v, dst1_v, semx1)
                load_idx(k0 + 3, src1_v, dst1_v)
                fire(src1_v, rowsa1_v, rowsb1_v, sem1)

        return carry

    lax.fori_loop(0, (nch + 1) // 2, body, 0)
    drain_scatter(rowsa0_v, rowsb0_v, dst0_v, semx0)
    drain_scatter(rowsa1_v, rowsb1_v, dst1_v, semx1)
    plsc.subcore_barrier()
    nout = NSEED // NS
    pltpu.sync_copy(acca_sh.at[pl.ds(s * nout, nout)],
                    pa_hbm.at[c, pl.ds(s * nout, nout)])
    pltpu.sync_copy(accb_sh.at[pl.ds(s * nout, nout)],
                    pb_hbm.at[c, pl.ds(s * nout, nout)])


def _tc1_body(x_ref, ids_ref, nt_ref, seed_ref, we_ref, wt_ref, wn_ref,
              ws_ref, b01_ref, b1_ref, t0_ref, t1_ref, hs_ref):
    x = x_ref[...]
    ids = ids_ref[...]
    iota = lax.broadcasted_iota(jnp.int32, (1, NSEED), 1)
    onehot = (ids == iota).astype(jnp.float32)
    st = jnp.dot(onehot, seed_ref[...], preferred_element_type=jnp.float32)
    rel = st - nt_ref[...]
    k = lax.broadcasted_iota(jnp.int32, (1, D // 2), 1).astype(jnp.float32)
    freqs = jnp.exp((-np.log(10000.0) / (D // 2)) * k)
    ang = rel * freqs
    pe = jnp.concatenate([jnp.sin(ang), jnp.cos(ang)], axis=1)
    h0 = (jnp.dot(x, we_ref[...], preferred_element_type=jnp.float32)
          + jnp.dot(pe, wt_ref[...], preferred_element_type=jnp.float32)
          + b01_ref[...])
    hn = jnp.dot(h0, wn_ref[...], preferred_element_type=jnp.float32)
    t0_ref[...] = hn[:, :H]
    t1_ref[...] = hn[:, H:]
    hs_ref[...] = (jnp.dot(h0, ws_ref[...], preferred_element_type=jnp.float32)
                   + b1_ref[...])


def _tc1(x, ids2d, nt2d, seed2d, W_enc, W_time, W_neigh1, W_self1, b01, b1):
    R = 1000
    grid = (N // R,)
    return pl.pallas_call(
        _tc1_body,
        grid=grid,
        in_specs=[
            pl.BlockSpec((R, D), lambda i: (i, 0)),
            pl.BlockSpec((R, 1), lambda i: (i, 0)),
            pl.BlockSpec((R, 1), lambda i: (i, 0)),
            pl.BlockSpec((NSEED, 1), lambda i: (0, 0)),
            pl.BlockSpec((D, D), lambda i: (0, 0)),
            pl.BlockSpec((D, D), lambda i: (0, 0)),
            pl.BlockSpec((D, D), lambda i: (0, 0)),
            pl.BlockSpec((D, D), lambda i: (0, 0)),
            pl.BlockSpec((1, D), lambda i: (0, 0)),
            pl.BlockSpec((1, D), lambda i: (0, 0)),
        ],
        out_specs=[
            pl.BlockSpec((R, H), lambda i: (i, 0)),
            pl.BlockSpec((R, H), lambda i: (i, 0)),
            pl.BlockSpec((R, D), lambda i: (i, 0)),
        ],
        out_shape=[
            jax.ShapeDtypeStruct((N, H), jnp.float32),
            jax.ShapeDtypeStruct((N, H), jnp.float32),
            jax.ShapeDtypeStruct((N, D), jnp.float32),
        ],
        compiler_params=pltpu.CompilerParams(
            dimension_semantics=("parallel",)),
    )(x, ids2d, nt2d, seed2d, W_enc, W_time, W_neigh1, W_self1, b01, b1)


def _tc2_body(hs_ref, a0_ref, a1_ref, d0_ref, d1_ref, wn2_ref,
              h1_ref, na_ref, nb_ref):
    deg = jnp.maximum(d0_ref[...] + d1_ref[...], 1.0)
    agg = jnp.concatenate([a0_ref[...], a1_ref[...]], axis=1) / deg
    h1 = jnp.maximum(hs_ref[...] + agg, 0.0)
    h1_ref[...] = h1
    hn2 = jnp.dot(h1, wn2_ref[...], preferred_element_type=jnp.float32)
    na_ref[...] = hn2[:, :H]
    nb_ref[...] = hn2[:, H:]


def _tc2(hs1b, a0, a1, d0, d1, W_neigh2):
    R = 1000
    grid = (N // R,)
    return pl.pallas_call(
        _tc2_body,
        grid=grid,
        in_specs=[
            pl.BlockSpec((R, D), lambda i: (i, 0)),
            pl.BlockSpec((R, H), lambda i: (i, 0)),
            pl.BlockSpec((R, H), lambda i: (i, 0)),
            pl.BlockSpec((R, 1), lambda i: (i, 0)),
            pl.BlockSpec((R, 1), lambda i: (i, 0)),
            pl.BlockSpec((D, D), lambda i: (0, 0)),
        ],
        out_specs=[
            pl.BlockSpec((R, D), lambda i: (i, 0)),
            pl.BlockSpec((R, H), lambda i: (i, 0)),
            pl.BlockSpec((R, H), lambda i: (i, 0)),
        ],
        out_shape=[
            jax.ShapeDtypeStruct((N, D), jnp.float32),
            jax.ShapeDtypeStruct((N, H), jnp.float32),
            jax.ShapeDtypeStruct((N, H), jnp.float32),
        ],
        compiler_params=pltpu.CompilerParams(
            dimension_semantics=("parallel",)),
    )(hs1b, a0, a1, d0, d1, W_neigh2)


def _tc3_body(h1s_ref, pa0_ref, pa1_ref, pb0_ref, pb1_ref, d0_ref, d1_ref,
              ws2_ref, b2_ref, wm1_ref, bm1_ref, g_ref, bln_ref, wm2_ref,
              bm2_ref, out_ref):
    deg = jnp.maximum(d0_ref[...] + d1_ref[...], 1.0)
    agg = jnp.concatenate(
        [pa0_ref[...] + pa1_ref[...], pb0_ref[...] + pb1_ref[...]], axis=1) / deg
    h2 = jnp.maximum(
        jnp.dot(h1s_ref[...], ws2_ref[...], preferred_element_type=jnp.float32)
        + b2_ref[...] + agg, 0.0)
    z = (jnp.dot(h2, wm1_ref[...], preferred_element_type=jnp.float32)
         + bm1_ref[...])
    mu = jnp.mean(z, axis=1, keepdims=True)
    var = jnp.mean((z - mu) * (z - mu), axis=1, keepdims=True)
    z = (z - mu) * lax.rsqrt(var + 1e-5) * g_ref[...] + bln_ref[...]
    z = jnp.maximum(z, 0.0)
    out_ref[...] = (jnp.dot(z, wm2_ref[...], preferred_element_type=jnp.float32)
                    + bm2_ref[...])


def _tc3(h1s, pa0, pa1, pb0, pb1, d0_s, d1_s, W_self2, b2, W_mlp1, b_mlp1,
         ln_g, ln_b, W_mlp2, b_mlp2):
    return pl.pallas_call(
        _tc3_body,
        out_shape=jax.ShapeDtypeStruct((NSEED, 1), jnp.float32),
    )(h1s, pa0, pa1, pb0, pb1, d0_s, d1_s, W_self2, b2, W_mlp1, b_mlp1, ln_g,
      ln_b, W_mlp2, b_mlp2)


@jax.jit
def kernel(x, node_time, seed_time, W_enc, b_enc, W_time, b_time, W_self1,
           W_neigh1, b1, W_self2, W_neigh2, b2, W_mlp1, b_mlp1, ln_g, ln_b,
           W_mlp2, b_mlp2, batch_ids, edge_index):
    src = edge_index[0].astype(jnp.int32)
    dst = edge_index[1].astype(jnp.int32)

    src_p = jnp.concatenate([src, jnp.zeros((EPAD - E,), jnp.int32)])
    dst_p = jnp.concatenate([dst, jnp.full((EPAD - E,), N, jnp.int32)])
    zrow = jnp.zeros((ZROW, H), jnp.float32)
    onesr = jnp.ones((CE, H), jnp.float32)
    padz = jnp.zeros((PERW + 16,), jnp.int32)
    padd = jnp.full((PERW + 16,), NSEED, jnp.int32)

    degp, csrc, cdst, cnts = _deg_kernel(src_p, dst_p, zrow, onesr, padz, padd)

    b01 = (b_enc + b_time).reshape(1, D)
    t0, t1, hs1b = _tc1(x, batch_ids.astype(jnp.int32).reshape(N, 1),
                        node_time.reshape(N, 1), seed_time.reshape(NSEED, 1),
                        W_enc, W_time, W_neigh1, W_self1,
                        b01, b1.reshape(1, D))

    (agg,) = _agg1_kernel(src_p, dst_p, t0, t1, zrow)

    h1, na, nb = _tc2(hs1b, agg[0, :N], agg[1, :N],
                      degp[0, :N, 0:1], degp[1, :N, 0:1], W_neigh2)

    zrowb = jnp.zeros((ZROWB, H), jnp.float32)
    pa, pb = _agg2_kernel(csrc, cdst, cnts, na, nb, zrowb)

    out = _tc3(h1[:NSEED], pa[0], pa[1], pb[0], pb[1],
               degp[0, :NSEED, 0:1], degp[1, :NSEED, 0:1],
               W_self2, b2.reshape(1, D), W_mlp1, b_mlp1.reshape(1, H),
               ln_g.reshape(1, H), ln_b.reshape(1, H), W_mlp2,
               b_mlp2.reshape(1, 1))
    return out.reshape(NSEED)


# speedup vs baseline: 1.0196x; 1.0196x over previous
import functools

import jax
import jax.numpy as jnp
import numpy as np
from jax import lax
from jax.experimental import pallas as pl
from jax.experimental.pallas import tpu as pltpu
from jax.experimental.pallas import tpu_sc as plsc

N = 10000
E = 160000
D = 256
NSEED = 1024
H = 128

NC = 2
NS = 16
NW = NC * NS

EPAD = 163840
CE = 128
CEB = 80
NS1 = 4
CEC = 32
NSLOT = 4
NROW = 10112
ZROW = NROW // NS
BROW = 1040
ZROWB = BROW // NS

_mesh = plsc.VectorSubcoreMesh(
    core_axis_name="c", subcore_axis_name="s", num_cores=NC, num_subcores=NS)


@functools.partial(
    pl.kernel,
    out_type=(
        jax.ShapeDtypeStruct((NC, NROW, H), jnp.float32),
    ),
    mesh=_mesh,
    scratch_types=(
        [pltpu.VMEM_SHARED((NROW, H), jnp.float32)]
        + [pltpu.VMEM((CEB, H), jnp.float32)] * NS1
        + [pltpu.VMEM((CEB,), jnp.int32)] * (2 * NS1)
        + [pltpu.SemaphoreType.DMA] * (3 * NS1)
    ),
    compiler_params=pltpu.CompilerParams(needs_layout_passes=False),
)
def _agg1_kernel(src_hbm, dst_hbm, t0_hbm, t1_hbm, zrow_hbm,
                 agg_hbm, acc_sh, *rest):
    rows_v = rest[:NS1]
    src_v = rest[NS1:2 * NS1]
    dst_v = rest[2 * NS1:3 * NS1]
    sem_i = rest[3 * NS1:4 * NS1]
    sem_g = rest[4 * NS1:5 * NS1]
    sem_s = rest[5 * NS1:6 * NS1]
    c = lax.axis_index("c")
    s = lax.axis_index("s")
    pltpu.sync_copy(zrow_hbm, acc_sh.at[pl.ds(s * ZROW, ZROW)])
    plsc.subcore_barrier()

    nchunk = (EPAD // NS) // CEB
    ebase = s * (EPAD // NS)

    def srcat(k):
        return src_hbm.at[pl.ds(ebase + k * CEB, CEB)]

    def dstat(k):
        return dst_hbm.at[pl.ds(ebase + k * CEB, CEB)]

    def run(table_hbm):
        for b in range(NS1):
            pltpu.sync_copy(srcat(b), src_v[b])
            pltpu.sync_copy(dstat(b), dst_v[b])
            pltpu.async_copy(table_hbm.at[src_v[b]], rows_v[b], sem_g[b])

        def body(jj, carry):
            for b in range(NS1):
                pltpu.make_async_copy(table_hbm.at[src_v[b]], rows_v[b],
                                      sem_g[b]).wait()
                pltpu.async_copy(rows_v[b], acc_sh.at[dst_v[b]], sem_s[b],
                                 add=True)

            @pl.when(jj < nchunk // NS1 - 1)
            def _():
                for b in range(NS1):
                    k = NS1 * jj + b + NS1
                    pltpu.make_async_copy(rows_v[b], acc_sh.at[dst_v[b]],
                                          sem_s[b]).wait()
                    pltpu.async_copy(srcat(k), src_v[b], sem_i[b])
                    pltpu.async_copy(dstat(k), dst_v[b], sem_i[b])
                for b in range(NS1):
                    k = NS1 * jj + b + NS1
                    pltpu.make_async_copy(srcat(k), src_v[b], sem_i[b]).wait()
                    pltpu.make_async_copy(dstat(k), dst_v[b], sem_i[b]).wait()
                    pltpu.async_copy(table_hbm.at[src_v[b]], rows_v[b],
                                     sem_g[b])
            return carry

        lax.fori_loop(0, nchunk // NS1, body, 0)
        for b in range(NS1):
            pltpu.make_async_copy(rows_v[b], acc_sh.at[dst_v[b]],
                                  sem_s[b]).wait()

    @pl.when(c == 0)
    def _():
        run(t0_hbm)

    @pl.when(c == 1)
    def _():
        run(t1_hbm)

    plsc.subcore_barrier()
    pltpu.sync_copy(acc_sh.at[pl.ds(s * ZROW, ZROW)],
                    agg_hbm.at[c, pl.ds(s * ZROW, ZROW)])


PERW = EPAD // NW


@functools.partial(
    pl.kernel,
    out_type=(
        jax.ShapeDtypeStruct((NC, NROW, H), jnp.float32),
        jax.ShapeDtypeStruct((EPAD,), jnp.int32),
        jax.ShapeDtypeStruct((EPAD,), jnp.int32),
        jax.ShapeDtypeStruct((NW * 16,), jnp.int32),
    ),
    mesh=_mesh,
    scratch_types=(
        [pltpu.VMEM_SHARED((NROW, H), jnp.float32),
         pltpu.VMEM((CE, H), jnp.float32),
         pltpu.VMEM((PERW + 16,), jnp.int32),
         pltpu.VMEM((PERW + 16,), jnp.int32),
         pltpu.VMEM((16,), jnp.int32)]
        + [pltpu.VMEM((CE,), jnp.int32)] * (2 * NSLOT)
        + [pltpu.SemaphoreType.DMA] * (2 * NSLOT)
    ),
    compiler_params=pltpu.CompilerParams(needs_layout_passes=False),
)
def _deg_kernel(src_hbm, dst_hbm, zrow_hbm, ones_hbm, padz_hbm, padd_hbm,
                deg_hbm, csrc_hbm, cdst_hbm, cnt_hbm,
                acc_sh, ones_v, csrc_v, cdst_v, cnt_v, *rest):
    dst_v = rest[:NSLOT]
    src_v = rest[NSLOT:2 * NSLOT]
    sem_i = rest[2 * NSLOT:3 * NSLOT]
    sem_s = rest[3 * NSLOT:4 * NSLOT]
    c = lax.axis_index("c")
    s = lax.axis_index("s")
    w = s * NC + c
    pltpu.sync_copy(zrow_hbm, acc_sh.at[pl.ds(s * ZROW, ZROW)])
    pltpu.sync_copy(ones_hbm, ones_v)
    pltpu.sync_copy(padz_hbm, csrc_v.at[pl.ds(0, PERW + 16)])
    pltpu.sync_copy(padd_hbm, cdst_v.at[pl.ds(0, PERW + 16)])
    plsc.subcore_barrier()

    nchunk = PERW // CE
    ebase = w * PERW

    def didx_at(k):
        return dst_hbm.at[pl.ds(ebase + k * CE, CE)]

    def sidx_at(k):
        return src_hbm.at[pl.ds(ebase + k * CE, CE)]

    for b in range(NSLOT):
        pltpu.async_copy(didx_at(b), dst_v[b], sem_i[b])
        pltpu.async_copy(sidx_at(b), src_v[b], sem_i[b])

    def body(jj, cnt):
        for b in range(NSLOT):
            k = NSLOT * jj + b
            pltpu.make_async_copy(didx_at(k), dst_v[b], sem_i[b]).wait()
            pltpu.make_async_copy(sidx_at(k), src_v[b], sem_i[b]).wait()
            pltpu.async_copy(ones_v, acc_sh.at[dst_v[b]], sem_s[b], add=True)
            for g in range(CE // 16):
                dv = dst_v[b][pl.ds(g * 16, 16)]
                sv = src_v[b][pl.ds(g * 16, 16)]
                m = dv < NSEED
                dcl = jnp.where(m, dv, NSEED)
                dsrt, ssrt = plsc.sort_key_val(dcl, sv)
                cdst_v[pl.ds(cnt, 16)] = dsrt
                csrc_v[pl.ds(cnt, 16)] = ssrt
                cnt = cnt + jnp.sum(jnp.where(m, 1, 0))

        @pl.when(jj < nchunk // NSLOT - 1)
        def _():
            for b in range(NSLOT):
                k = NSLOT * jj + b
                pltpu.make_async_copy(ones_v, acc_sh.at[dst_v[b]],
                                      sem_s[b]).wait()
                pltpu.async_copy(didx_at(k + NSLOT), dst_v[b], sem_i[b])
                pltpu.async_copy(sidx_at(k + NSLOT), src_v[b], sem_i[b])
        return cnt

    cnt = lax.fori_loop(0, nchunk // NSLOT, body, jnp.int32(0))
    for b in range(NSLOT):
        pltpu.make_async_copy(ones_v, acc_sh.at[dst_v[b]], sem_s[b]).wait()
    pltpu.sync_copy(csrc_v.at[pl.ds(0, PERW)], csrc_hbm.at[pl.ds(ebase, PERW)])
    pltpu.sync_copy(cdst_v.at[pl.ds(0, PERW)], cdst_hbm.at[pl.ds(ebase, PERW)])
    nch = (cnt + (CEC - 1)) // CEC
    cnt_v[pl.ds(0, 16)] = jnp.broadcast_to(nch, (16,))
    pltpu.sync_copy(cnt_v, cnt_hbm.at[pl.ds(w * 16, 16)])
    plsc.subcore_barrier()
    pltpu.sync_copy(acc_sh.at[pl.ds(s * ZROW, ZROW)],
                    deg_hbm.at[c, pl.ds(s * ZROW, ZROW)])


@functools.partial(
    pl.kernel,
    out_type=(
        jax.ShapeDtypeStruct((NC, NSEED, H), jnp.float32),
        jax.ShapeDtypeStruct((NC, NSEED, H), jnp.float32),
    ),
    mesh=_mesh,
    scratch_types=(
        [pltpu.VMEM_SHARED((BROW, H), jnp.float32)] * 2
        + [pltpu.VMEM((CEC, H), jnp.float32)] * 4
        + [pltpu.VMEM((CEC,), jnp.int32)] * 4
        + [pltpu.VMEM((16,), jnp.int32)]
        + [pltpu.SemaphoreType.DMA] * 4
    ),
    compiler_params=pltpu.CompilerParams(needs_layout_passes=False),
)
def _agg2_kernel(src_hbm, dst_hbm, cnt_hbm, ta_hbm, tb_hbm, zrow_hbm,
                 pa_hbm, pb_hbm, acca_sh, accb_sh,
                 rowsa0_v, rowsb0_v, rowsa1_v, rowsb1_v,
                 src0_v, src1_v, dst0_v, dst1_v, cnt_v, sem0, sem1,
                 semx0, semx1):
    c = lax.axis_index("c")
    s = lax.axis_index("s")
    w = s * NC + c
    pltpu.sync_copy(zrow_hbm, acca_sh.at[pl.ds(s * ZROWB, ZROWB)])
    pltpu.sync_copy(zrow_hbm, accb_sh.at[pl.ds(s * ZROWB, ZROWB)])
    pltpu.sync_copy(cnt_hbm.at[pl.ds(w * 16, 16)], cnt_v)
    plsc.subcore_barrier()

    nch = jnp.maximum(jnp.max(cnt_v[...]), 2)
    ebase = w * PERW

    def load_idx(k, src_v, dst_v):
        base = ebase + k * CEC
        pltpu.sync_copy(src_hbm.at[pl.ds(base, CEC)], src_v)
        pltpu.sync_copy(dst_hbm.at[pl.ds(base, CEC)], dst_v)

    def fire(src_v, ra, rb, sem):
        pltpu.async_copy(ta_hbm.at[src_v], ra, sem)
        pltpu.async_copy(tb_hbm.at[src_v], rb, sem)

    def drain_scatter(ra, rb, dst_v, sem):
        pltpu.make_async_copy(ra, acca_sh.at[dst_v], sem).wait()
        pltpu.make_async_copy(rb, accb_sh.at[dst_v], sem).wait()

    load_idx(0, src0_v, dst0_v)
    fire(src0_v, rowsa0_v, rowsb0_v, sem0)
    load_idx(1, src1_v, dst1_v)
    fire(src1_v, rowsa1_v, rowsb1_v, sem1)

    def body(j, carry):
        k0 = 2 * j
        pltpu.make_async_copy(ta_hbm.at[src0_v], rowsa0_v, sem0).wait()
        pltpu.make_async_copy(tb_hbm.at[src0_v], rowsb0_v, sem0).wait()
        pltpu.async_copy(rowsa0_v, acca_sh.at[dst0_v], semx0, add=True)
        pltpu.async_copy(rowsb0_v, accb_sh.at[dst0_v], semx0, add=True)

        @pl.when(k0 + 2 < nch)
        def _():
            drain_scatter(rowsa0_v, rowsb0_v, dst0_v, semx0)
            load_idx(k0 + 2, src0_v, dst0_v)
            fire(src0_v, rowsa0_v, rowsb0_v, sem0)

        @pl.when(k0 + 1 < nch)
        def _():
            pltpu.make_async_copy(ta_hbm.at[src1_v], rowsa1_v, sem1).wait()
            pltpu.make_async_copy(tb_hbm.at[src1_v], rowsb1_v, sem1).wait()
            pltpu.async_copy(rowsa1_v, acca_sh.at[dst1_v], semx1, add=True)
            pltpu.async_copy(rowsb1_v, accb_sh.at[dst1_v], semx1, add=True)

            @pl.when(k0 + 3 < nch)
            def _():
                drain_scatter(rowsa1_v, rowsb1_v, dst1_v, semx1)
                load_idx(k0 + 3, src1_v, dst1_v)
                fire(src1_v, rowsa1_v, rowsb1_v, sem1)

        return carry

    lax.fori_loop(0, (nch + 1) // 2, body, 0)
    drain_scatter(rowsa0_v, rowsb0_v, dst0_v, semx0)
    drain_scatter(rowsa1_v, rowsb1_v, dst1_v, semx1)
    plsc.subcore_barrier()
    nout = NSEED // NS
    pltpu.sync_copy(acca_sh.at[pl.ds(s * nout, nout)],
                    pa_hbm.at[c, pl.ds(s * nout, nout)])
    pltpu.sync_copy(accb_sh.at[pl.ds(s * nout, nout)],
                    pb_hbm.at[c, pl.ds(s * nout, nout)])


def _tc1_body(x_ref, ids_ref, nt_ref, seed_ref, we_ref, wt_ref, wn_ref,
              ws_ref, b01_ref, b1_ref, t0_ref, t1_ref, hs_ref):
    x = x_ref[...]
    ids = ids_ref[...]
    iota = lax.broadcasted_iota(jnp.int32, (1, NSEED), 1)
    onehot = (ids == iota).astype(jnp.float32)
    st = jnp.dot(onehot, seed_ref[...], preferred_element_type=jnp.float32)
    rel = st - nt_ref[...]
    k = lax.broadcasted_iota(jnp.int32, (1, D // 2), 1).astype(jnp.float32)
    freqs = jnp.exp((-np.log(10000.0) / (D // 2)) * k)
    ang = rel * freqs
    pe = jnp.concatenate([jnp.sin(ang), jnp.cos(ang)], axis=1)
    h0 = (jnp.dot(x, we_ref[...], preferred_element_type=jnp.float32)
          + jnp.dot(pe, wt_ref[...], preferred_element_type=jnp.float32)
          + b01_ref[...])
    hn = jnp.dot(h0, wn_ref[...], preferred_element_type=jnp.float32)
    t0_ref[...] = hn[:, :H]
    t1_ref[...] = hn[:, H:]
    hs_ref[...] = (jnp.dot(h0, ws_ref[...], preferred_element_type=jnp.float32)
                   + b1_ref[...])


def _tc1(x, ids2d, nt2d, seed2d, W_enc, W_time, W_neigh1, W_self1, b01, b1):
    R = 1000
    grid = (N // R,)
    return pl.pallas_call(
        _tc1_body,
        grid=grid,
        in_specs=[
            pl.BlockSpec((R, D), lambda i: (i, 0)),
            pl.BlockSpec((R, 1), lambda i: (i, 0)),
            pl.BlockSpec((R, 1), lambda i: (i, 0)),
            pl.BlockSpec((NSEED, 1), lambda i: (0, 0)),
            pl.BlockSpec((D, D), lambda i: (0, 0)),
            pl.BlockSpec((D, D), lambda i: (0, 0)),
            pl.BlockSpec((D, D), lambda i: (0, 0)),
            pl.BlockSpec((D, D), lambda i: (0, 0)),
            pl.BlockSpec((1, D), lambda i: (0, 0)),
            pl.BlockSpec((1, D), lambda i: (0, 0)),
        ],
        out_specs=[
            pl.BlockSpec((R, H), lambda i: (i, 0)),
            pl.BlockSpec((R, H), lambda i: (i, 0)),
            pl.BlockSpec((R, D), lambda i: (i, 0)),
        ],
        out_shape=[
            jax.ShapeDtypeStruct((N, H), jnp.float32),
            jax.ShapeDtypeStruct((N, H), jnp.float32),
            jax.ShapeDtypeStruct((N, D), jnp.float32),
        ],
        compiler_params=pltpu.CompilerParams(
            dimension_semantics=("parallel",)),
    )(x, ids2d, nt2d, seed2d, W_enc, W_time, W_neigh1, W_self1, b01, b1)


def _tc2_body(hs_ref, a0_ref, a1_ref, d0_ref, d1_ref, wn2_ref,
              h1_ref, na_ref, nb_ref):
    deg = jnp.maximum(d0_ref[...] + d1_ref[...], 1.0)
    agg = jnp.concatenate([a0_ref[...], a1_ref[...]], axis=1) / deg
    h1 = jnp.maximum(hs_ref[...] + agg, 0.0)
    h1_ref[...] = h1
    hn2 = jnp.dot(h1, wn2_ref[...], preferred_element_type=jnp.float32)
    na_ref[...] = hn2[:, :H]
    nb_ref[...] = hn2[:, H:]


def _tc2(hs1b, a0, a1, d0, d1, W_neigh2):
    R = 1000
    grid = (N // R,)
    return pl.pallas_call(
        _tc2_body,
        grid=grid,
        in_specs=[
            pl.BlockSpec((R, D), lambda i: (i, 0)),
            pl.BlockSpec((R, H), lambda i: (i, 0)),
            pl.BlockSpec((R, H), lambda i: (i, 0)),
            pl.BlockSpec((R, 1), lambda i: (i, 0)),
            pl.BlockSpec((R, 1), lambda i: (i, 0)),
            pl.BlockSpec((D, D), lambda i: (0, 0)),
        ],
        out_specs=[
            pl.BlockSpec((R, D), lambda i: (i, 0)),
            pl.BlockSpec((R, H), lambda i: (i, 0)),
            pl.BlockSpec((R, H), lambda i: (i, 0)),
        ],
        out_shape=[
            jax.ShapeDtypeStruct((N, D), jnp.float32),
            jax.ShapeDtypeStruct((N, H), jnp.float32),
            jax.ShapeDtypeStruct((N, H), jnp.float32),
        ],
        compiler_params=pltpu.CompilerParams(
            dimension_semantics=("parallel",)),
    )(hs1b, a0, a1, d0, d1, W_neigh2)


def _tc3_body(h1s_ref, pa0_ref, pa1_ref, pb0_ref, pb1_ref, d0_ref, d1_ref,
              ws2_ref, b2_ref, wm1_ref, bm1_ref, g_ref, bln_ref, wm2_ref,
              bm2_ref, out_ref):
    deg = jnp.maximum(d0_ref[...] + d1_ref[...], 1.0)
    agg = jnp.concatenate(
        [pa0_ref[...] + pa1_ref[...], pb0_ref[...] + pb1_ref[...]], axis=1) / deg
    h2 = jnp.maximum(
        jnp.dot(h1s_ref[...], ws2_ref[...], preferred_element_type=jnp.float32)
        + b2_ref[...] + agg, 0.0)
    z = (jnp.dot(h2, wm1_ref[...], preferred_element_type=jnp.float32)
         + bm1_ref[...])
    mu = jnp.mean(z, axis=1, keepdims=True)
    var = jnp.mean((z - mu) * (z - mu), axis=1, keepdims=True)
    z = (z - mu) * lax.rsqrt(var + 1e-5) * g_ref[...] + bln_ref[...]
    z = jnp.maximum(z, 0.0)
    out_ref[...] = (jnp.dot(z, wm2_ref[...], preferred_element_type=jnp.float32)
                    + bm2_ref[...])


def _tc3(h1s, pa0, pa1, pb0, pb1, d0_s, d1_s, W_self2, b2, W_mlp1, b_mlp1,
         ln_g, ln_b, W_mlp2, b_mlp2):
    return pl.pallas_call(
        _tc3_body,
        out_shape=jax.ShapeDtypeStruct((NSEED, 1), jnp.float32),
    )(h1s, pa0, pa1, pb0, pb1, d0_s, d1_s, W_self2, b2, W_mlp1, b_mlp1, ln_g,
      ln_b, W_mlp2, b_mlp2)


@jax.jit
def kernel(x, node_time, seed_time, W_enc, b_enc, W_time, b_time, W_self1,
           W_neigh1, b1, W_self2, W_neigh2, b2, W_mlp1, b_mlp1, ln_g, ln_b,
           W_mlp2, b_mlp2, batch_ids, edge_index):
    src = edge_index[0].astype(jnp.int32)
    dst = edge_index[1].astype(jnp.int32)

    src_p = jnp.concatenate([src, jnp.zeros((EPAD - E,), jnp.int32)])
    dst_p = jnp.concatenate([dst, jnp.full((EPAD - E,), N, jnp.int32)])
    zrow = jnp.zeros((ZROW, H), jnp.float32)
    onesr = jnp.ones((CE, H), jnp.float32)
    padz = jnp.zeros((PERW + 16,), jnp.int32)
    padd = jnp.full((PERW + 16,), NSEED, jnp.int32)

    degp, csrc, cdst, cnts = _deg_kernel(src_p, dst_p, zrow, onesr, padz, padd)

    b01 = (b_enc + b_time).reshape(1, D)
    t0, t1, hs1b = _tc1(x, batch_ids.astype(jnp.int32).reshape(N, 1),
                        node_time.reshape(N, 1), seed_time.reshape(NSEED, 1),
                        W_enc, W_time, W_neigh1, W_self1,
                        b01, b1.reshape(1, D))

    (agg,) = _agg1_kernel(src_p, dst_p, t0, t1, zrow)

    h1, na, nb = _tc2(hs1b, agg[0, :N], agg[1, :N],
                      degp[0, :N, 0:1], degp[1, :N, 0:1], W_neigh2)

    zrowb = jnp.zeros((ZROWB, H), jnp.float32)
    pa, pb = _agg2_kernel(csrc, cdst, cnts, na, nb, zrowb)

    out = _tc3(h1[:NSEED], pa[0], pa[1], pb[0], pb[1],
               degp[0, :NSEED, 0:1], degp[1, :NSEED, 0:1],
               W_self2, b2.reshape(1, D), W_mlp1, b_mlp1.reshape(1, H),
               ln_g.reshape(1, H), ln_b.reshape(1, H), W_mlp2,
               b_mlp2.reshape(1, 1))
    return out.reshape(NSEED)
